# Initial kernel scaffold; baseline (speedup 1.0000x reference)
import jax
import jax.numpy as jnp
from jax.experimental import pallas as pl


def kernel(x, gate_w, gate_b, w1, b1, w2, b2):
    raise NotImplementedError("write your pallas kernel here")


# speedup vs baseline: 1.0778x; 1.0778x over previous
import jax
import jax.numpy as jnp
from jax.experimental import pallas as pl
from jax.experimental.pallas import tpu as pltpu

_NUM_EXPERTS = 8
_TOP_K = 2
_D_IN = 768
_D_OUT = 768
_D_FF = 4 * _D_IN
_N_TOK = 2048
_CH = 768


def _moe_body(x_ref, gw_ref, gb_ref, w1_ref, b1_ref, w2_ref, b2_ref,
              out_ref, comb_ref):
    e = pl.program_id(0)
    c = pl.program_id(1)
    n = x_ref.shape[0]

    @pl.when((e == 0) & (c == 0))
    def _router():
        xv = x_ref[...]
        logits = jax.lax.dot_general(
            xv.astype(jnp.bfloat16), gw_ref[...].astype(jnp.bfloat16),
            (((1,), (1,)), ((), ())),
            preferred_element_type=jnp.float32) + gb_ref[...]
        col = jax.lax.broadcasted_iota(jnp.int32, (n, _NUM_EXPERTS), 1)
        m1 = jnp.max(logits, axis=1, keepdims=True)
        i1 = jnp.min(jnp.where(logits == m1, col, _NUM_EXPERTS),
                     axis=1, keepdims=True)
        mask1 = col == i1
        logits2 = jnp.where(mask1, -1e30, logits)
        m2 = jnp.max(logits2, axis=1, keepdims=True)
        i2 = jnp.min(jnp.where(logits2 == m2, col, _NUM_EXPERTS),
                     axis=1, keepdims=True)
        mask2 = col == i2
        e21 = jnp.exp(m2 - m1)
        w_hi = 1.0 / (1.0 + e21)
        comb_ref[...] = (jnp.where(mask1, w_hi, 0.0)
                         + jnp.where(mask2, 1.0 - w_hi, 0.0))

    xv = x_ref[...].astype(jnp.bfloat16)
    h = jax.lax.dot_general(xv, w1_ref[0].astype(jnp.bfloat16),
                            (((1,), (1,)), ((), ())),
                            preferred_element_type=jnp.float32)
    h = jnp.maximum(h + b1_ref[pl.ds(e, 1), :], 0.0)
    y = jax.lax.dot_general(h.astype(jnp.bfloat16),
                            w2_ref[0].astype(jnp.bfloat16),
                            (((1,), (1,)), ((), ())),
                            preferred_element_type=jnp.float32)
    onehot = (jax.lax.broadcasted_iota(jnp.int32, (_NUM_EXPERTS, 1), 0)
              == e).astype(jnp.float32)
    cvec = jnp.dot(comb_ref[...], onehot,
                   preferred_element_type=jnp.float32)
    add = cvec * y + (c == 0).astype(jnp.float32) * (cvec * b2_ref[pl.ds(e, 1), :])

    @pl.when((e == 0) & (c == 0))
    def _init():
        out_ref[...] = add

    @pl.when(~((e == 0) & (c == 0)))
    def _acc():
        out_ref[...] += add


def kernel(x, gate_w, gate_b, w1, b1, w2, b2):
    n = x.shape[0]
    gate_b2d = gate_b.reshape(1, _NUM_EXPERTS)
    grid = (_NUM_EXPERTS, _D_FF // _CH)
    return pl.pallas_call(
        _moe_body,
        grid=grid,
        in_specs=[
            pl.BlockSpec((n, _D_IN), lambda e, c: (0, 0)),
            pl.BlockSpec((_NUM_EXPERTS, _D_IN), lambda e, c: (0, 0)),
            pl.BlockSpec((1, _NUM_EXPERTS), lambda e, c: (0, 0)),
            pl.BlockSpec((1, _CH, _D_IN), lambda e, c: (e, c, 0)),
            pl.BlockSpec((_NUM_EXPERTS, _CH), lambda e, c: (0, c)),
            pl.BlockSpec((1, _D_OUT, _CH), lambda e, c: (e, 0, c)),
            pl.BlockSpec((_NUM_EXPERTS, _D_OUT), lambda e, c: (0, 0)),
        ],
        out_specs=pl.BlockSpec((n, _D_OUT), lambda e, c: (0, 0)),
        out_shape=jax.ShapeDtypeStruct((n, _D_OUT), jnp.float32),
        scratch_shapes=[pltpu.VMEM((n, _NUM_EXPERTS), jnp.float32)],
        compiler_params=pltpu.CompilerParams(
            dimension_semantics=("arbitrary", "arbitrary")),
    )(x, gate_w, gate_b2d, w1, b1, w2, b2)
